# baseline (speedup 1.0000x reference)
import jax
import jax.numpy as jnp
from jax.experimental import pallas as pl
from jax.experimental.pallas import tpu as pltpu

_XC = 10
_XR = 1000
_EC = 20
_EW = 16000


def _copy_body(x_ref, ea_ref, xo_ref, eo_ref, xbuf, ebuf, sxi, sxo, sei, seo):
    def xin(i):
        sl = pl.ds(i * _XR, _XR)
        return pltpu.make_async_copy(x_ref.at[sl, :], xbuf.at[sl, :], sxi.at[i])

    def xout(i):
        sl = pl.ds(i * _XR, _XR)
        return pltpu.make_async_copy(xbuf.at[sl, :], xo_ref.at[sl, :], sxo.at[i])

    def ein(i):
        sl = pl.ds(i * _EW, _EW)
        return pltpu.make_async_copy(ea_ref.at[:, sl], ebuf.at[:, sl], sei.at[i])

    def eout(i):
        sl = pl.ds(i * _EW, _EW)
        return pltpu.make_async_copy(ebuf.at[:, sl], eo_ref.at[:, sl], seo.at[i])

    for i in range(_XC):
        xin(i).start()
    for i in range(_EC):
        ein(i).start()
    for i in range(_XC):
        xin(i).wait()
        xout(i).start()
    for i in range(_EC):
        ein(i).wait()
        eout(i).start()
    for i in range(_XC):
        xout(i).wait()
    for i in range(_EC):
        eout(i).wait()


def kernel(x, edge_index, edge_attr):
    ea_t = edge_attr.T
    x_out, ea_out_t = pl.pallas_call(
        _copy_body,
        out_shape=(
            jax.ShapeDtypeStruct((10000, 128), x.dtype),
            jax.ShapeDtypeStruct((16, 320000), edge_attr.dtype),
        ),
        in_specs=[
            pl.BlockSpec(memory_space=pl.ANY),
            pl.BlockSpec(memory_space=pl.ANY),
        ],
        out_specs=(
            pl.BlockSpec(memory_space=pl.ANY),
            pl.BlockSpec(memory_space=pl.ANY),
        ),
        scratch_shapes=[
            pltpu.MemorySpace.VMEM((10000, 128), jnp.float32),
            pltpu.MemorySpace.VMEM((16, 320000), jnp.float32),
            pltpu.SemaphoreType.DMA((_XC,)),
            pltpu.SemaphoreType.DMA((_XC,)),
            pltpu.SemaphoreType.DMA((_EC,)),
            pltpu.SemaphoreType.DMA((_EC,)),
        ],
    )(x, ea_t)
    return (x_out, ea_out_t.T)


# speedup vs baseline: 1.0835x; 1.0835x over previous
import jax
from jax.experimental import pallas as pl

_GRID = 2
_XB = 10000 // _GRID
_EB = 320000 // _GRID


def _copy_body(x_ref, ea_ref, xo_ref, eo_ref):
    xo_ref[...] = x_ref[...]
    eo_ref[...] = ea_ref[...]


def kernel(x, edge_index, edge_attr):
    ea_t = edge_attr.T
    x_out, ea_out_t = pl.pallas_call(
        _copy_body,
        grid=(_GRID,),
        out_shape=(
            jax.ShapeDtypeStruct((10000, 128), x.dtype),
            jax.ShapeDtypeStruct((16, 320000), edge_attr.dtype),
        ),
        in_specs=[
            pl.BlockSpec((_XB, 128), lambda i: (i, 0)),
            pl.BlockSpec((16, _EB), lambda i: (0, i)),
        ],
        out_specs=(
            pl.BlockSpec((_XB, 128), lambda i: (i, 0)),
            pl.BlockSpec((16, _EB), lambda i: (0, i)),
        ),
    )(x, ea_t)
    return (x_out, ea_out_t.T)
